# Initial kernel scaffold; baseline (speedup 1.0000x reference)
import jax
import jax.numpy as jnp
from jax.experimental import pallas as pl


def kernel(x, edge_index, edge_attr, Wv0, We0, Wu0, bu0, Wv1, We1, Wu1, bu1, Wmix, bmix, Wih, Whh, bih, bhh, Wro, bro):
    raise NotImplementedError("write your pallas kernel here")


# speedup vs baseline: 5.6339x; 5.6339x over previous
import functools

import jax
import jax.numpy as jnp
from jax import lax
from jax.experimental import pallas as pl
from jax.experimental.pallas import tpu as pltpu
from jax.experimental.pallas import tpu_sc as plsc

N_ = 10000
E_ = 320000
HID = 128
HALF = 64
TWO_N = 2 * N_
TBL = 20480
TRASH = TWO_N
IDXR = 2560
EPAD = IDXR * 128
NTILES = 16
ROWS_PER_TILE = IDXR // NTILES
KROWS = 2
NCHUNK = ROWS_PER_TILE // KROWS
ZROWS = TBL // NTILES
CHUNK_E = KROWS * 128


def _idx_body(row_ref, col_ref, attr_ref, g_ref, s_ref):
    a = attr_ref[...]
    g_ref[...] = a * N_ + row_ref[...]
    s_ref[...] = a * N_ + col_ref[...]


def _make_idx(row2, col2, attr2):
    bs = pl.BlockSpec((256, 128), lambda i: (i, 0))
    return pl.pallas_call(
        _idx_body,
        grid=(10,),
        in_specs=[bs, bs, bs],
        out_specs=[bs, bs],
        out_shape=[jax.ShapeDtypeStruct((IDXR, 128), jnp.int32)] * 2,
    )(row2, col2, attr2)


def _proj_body(x_ref, w0_ref, w1_ref, lo_ref, hi_ref):
    i = pl.program_id(0)
    w = jnp.where(i < 10, w0_ref[...], w1_ref[...])
    p = jnp.dot(x_ref[...], w, preferred_element_type=jnp.float32)
    lo_ref[...] = p[:, :HALF]
    hi_ref[...] = p[:, HALF:]


def _proj(x, Wv0, Wv1):
    wspec = pl.BlockSpec((HID, HID), lambda i: (0, 0))
    ospec = pl.BlockSpec((1000, HALF), lambda i: (i, 0))
    return pl.pallas_call(
        _proj_body,
        grid=(20,),
        in_specs=[pl.BlockSpec((1000, HID), lambda i: (i % 10, 0)), wspec, wspec],
        out_specs=[ospec, ospec],
        out_shape=[jax.ShapeDtypeStruct((TBL, HALF), jnp.float32)] * 2,
    )(x, Wv0, Wv1)


def _seg_body(with_counts, *refs):
    if with_counts:
        (tlo, thi, gix, six, z64, z16, o16,
         out_lo, out_hi, cnt_a, cnt_b,
         gbuf, sbuf, rows, ones_v, acc, cnt, sem) = refs
    else:
        (tlo, thi, gix, six, z64,
         out_lo, out_hi,
         gbuf, sbuf, rows, acc, sem) = refs
        z16 = o16 = cnt_a = cnt_b = ones_v = cnt = None

    c = lax.axis_index("c")
    s = lax.axis_index("s")
    zsl = pl.ds(s * ZROWS, ZROWS)

    pltpu.sync_copy(z64, acc.at[zsl])
    if with_counts:
        pltpu.sync_copy(z16, cnt.at[zsl])
        pltpu.sync_copy(o16, ones_v)
    plsc.subcore_barrier()

    base = s * ROWS_PER_TILE

    def chunk(i, carry):
        r0 = base + i * KROWS
        pltpu.sync_copy(gix.at[pl.ds(r0, KROWS)], gbuf)
        pltpu.sync_copy(six.at[pl.ds(r0, KROWS)], sbuf)

        @pl.when(c == 0)
        def _():
            ds = [pltpu.async_copy(tlo.at[gbuf.at[j]],
                                   rows.at[pl.ds(j * 128, 128)], sem)
                  for j in range(KROWS)]
            for d in ds:
                d.wait()

        @pl.when(c == 1)
        def _():
            ds = [pltpu.async_copy(thi.at[gbuf.at[j]],
                                   rows.at[pl.ds(j * 128, 128)], sem)
                  for j in range(KROWS)]
            for d in ds:
                d.wait()

        for j in range(KROWS):
            pltpu.sync_copy(rows.at[pl.ds(j * 128, 128)],
                            acc.at[sbuf.at[j]], add=True)

        if with_counts:
            @pl.when(c == 0)
            def _():
                for j in range(KROWS):
                    pltpu.sync_copy(ones_v, cnt.at[sbuf.at[j]], add=True)

            @pl.when(c == 1)
            def _():
                for j in range(KROWS):
                    pltpu.sync_copy(ones_v, cnt.at[gbuf.at[j]], add=True)

        return carry

    lax.fori_loop(0, NCHUNK, chunk, 0)
    plsc.subcore_barrier()

    @pl.when(c == 0)
    def _():
        pltpu.sync_copy(acc.at[zsl], out_lo.at[zsl])
        if with_counts:
            pltpu.sync_copy(cnt.at[zsl], cnt_a.at[zsl])

    @pl.when(c == 1)
    def _():
        pltpu.sync_copy(acc.at[zsl], out_hi.at[zsl])
        if with_counts:
            pltpu.sync_copy(cnt.at[zsl], cnt_b.at[zsl])


def _make_segsum(with_counts):
    mesh = plsc.VectorSubcoreMesh(core_axis_name="c", subcore_axis_name="s")
    out_type = [jax.ShapeDtypeStruct((TBL, HALF), jnp.float32)] * 2
    scratch = [
        pltpu.VMEM((KROWS, 128), jnp.int32),
        pltpu.VMEM((KROWS, 128), jnp.int32),
        pltpu.VMEM((CHUNK_E, HALF), jnp.float32),
    ]
    if with_counts:
        out_type = out_type + [jax.ShapeDtypeStruct((TBL, 16), jnp.float32)] * 2
        scratch.append(pltpu.VMEM((128, 16), jnp.float32))
    scratch.append(pltpu.VMEM_SHARED((TBL, HALF), jnp.float32))
    if with_counts:
        scratch.append(pltpu.VMEM_SHARED((TBL, 16), jnp.float32))
    scratch.append(pltpu.SemaphoreType.DMA)
    return pl.kernel(
        functools.partial(_seg_body, with_counts),
        out_type=out_type,
        mesh=mesh,
        scratch_types=scratch,
        compiler_params=pltpu.CompilerParams(use_tc_tiling_on_sc=False),
    )


def _eproj_body(sl_ref, sh_ref, cnt_ref, w0_ref, w1_ref, lo_ref, hi_ref):
    i = pl.program_id(0)
    cnt = jnp.maximum(cnt_ref[...][:, :1], 1.0)
    feat = jnp.concatenate([sl_ref[...], sh_ref[...]], axis=1) / cnt
    w = jnp.where(i < 10, w0_ref[...], w1_ref[...])
    p = jnp.dot(feat, w, preferred_element_type=jnp.float32)
    lo_ref[...] = p[:, :HALF]
    hi_ref[...] = p[:, HALF:]


def _eproj(S_lo, S_hi, cnt_col, We0, We1):
    hspec = pl.BlockSpec((1000, HALF), lambda i: (i, 0))
    wspec = pl.BlockSpec((HID, HID), lambda i: (0, 0))
    return pl.pallas_call(
        _eproj_body,
        grid=(20,),
        in_specs=[hspec, hspec, pl.BlockSpec((1000, 16), lambda i: (i, 0)),
                  wspec, wspec],
        out_specs=[hspec, hspec],
        out_shape=[jax.ShapeDtypeStruct((TBL, HALF), jnp.float32)] * 2,
    )(S_lo, S_hi, cnt_col, We0, We1)


def _final_body(xl0, xh0, xl1, xh1, rl0, rh0, rl1, rh1, c0, c1,
                wu0, bu0, wu1, bu1, wm, bm, wih, bih, bhh, wro, bro,
                hn_ref, o_ref):
    f32 = jnp.float32
    xp0 = jnp.concatenate([xl0[...], xh0[...]], axis=1)
    xp1 = jnp.concatenate([xl1[...], xh1[...]], axis=1)
    na0 = jnp.concatenate([rl0[...], rh0[...]], axis=1) / jnp.maximum(c0[...][:, :1], 1.0)
    na1 = jnp.concatenate([rl1[...], rh1[...]], axis=1) / jnp.maximum(c1[...][:, :1], 1.0)
    wu0v = wu0[...]
    wu1v = wu1[...]
    h0 = jax.nn.relu(jnp.dot(xp0, wu0v[:HID], preferred_element_type=f32)
                     + jnp.dot(na0, wu0v[HID:], preferred_element_type=f32)
                     + bu0[...])
    h1 = jax.nn.relu(jnp.dot(xp1, wu1v[:HID], preferred_element_type=f32)
                     + jnp.dot(na1, wu1v[HID:], preferred_element_type=f32)
                     + bu1[...])
    wmv = wm[...]
    h = jax.nn.relu(jnp.dot(h0, wmv[:HID], preferred_element_type=f32)
                    + jnp.dot(h1, wmv[HID:], preferred_element_type=f32)
                    + bm[...])
    gi = jnp.dot(h, wih[...], preferred_element_type=f32) + bih[...]
    bhhv = bhh[...]
    r = jax.nn.sigmoid(gi[:, :HID] + bhhv[:, :HID])
    z = jax.nn.sigmoid(gi[:, HID:2 * HID] + bhhv[:, HID:2 * HID])
    n = jnp.tanh(gi[:, 2 * HID:] + r * bhhv[:, 2 * HID:])
    hn = (1.0 - z) * n
    hn_ref[...] = hn
    o_ref[...] = jnp.dot(hn, wro[...], preferred_element_type=f32) + bro[...]


def _final(X_lo, X_hi, R_lo, R_hi, cnt_row,
           Wu0, bu0, Wu1, bu1, Wmix, bmix, Wih, bih, bhh, Wro, bro):
    def hblk(off):
        return pl.BlockSpec((1000, HALF), lambda i, o=off: (i + o, 0))

    def cblk(off):
        return pl.BlockSpec((1000, 16), lambda i, o=off: (i + o, 0))

    def whole(a):
        return pl.BlockSpec(a.shape, lambda i: tuple(0 for _ in a.shape))

    ospec = pl.BlockSpec((1000, HID), lambda i: (i, 0))
    return pl.pallas_call(
        _final_body,
        grid=(10,),
        in_specs=[hblk(0), hblk(0), hblk(10), hblk(10),
                  hblk(0), hblk(0), hblk(10), hblk(10),
                  cblk(0), cblk(10),
                  whole(Wu0), whole(bu0), whole(Wu1), whole(bu1),
                  whole(Wmix), whole(bmix), whole(Wih), whole(bih),
                  whole(bhh), whole(Wro), whole(bro)],
        out_specs=[ospec, ospec],
        out_shape=[jax.ShapeDtypeStruct((N_, HID), jnp.float32)] * 2,
    )(X_lo, X_hi, X_lo, X_hi, R_lo, R_hi, R_lo, R_hi, cnt_row, cnt_row,
      Wu0, bu0, Wu1, bu1, Wmix, bmix, Wih, bih, bhh, Wro, bro)


def kernel(x, edge_index, edge_attr, Wv0, We0, Wu0, bu0, Wv1, We1, Wu1, bu1,
           Wmix, bmix, Wih, Whh, bih, bhh, Wro, bro):
    del Whh
    row = edge_index[0].astype(jnp.int32)
    col = edge_index[1].astype(jnp.int32)
    attr = edge_attr.astype(jnp.int32)
    pad = EPAD - E_
    row2 = jnp.pad(row, (0, pad), constant_values=N_).reshape(IDXR, 128)
    col2 = jnp.pad(col, (0, pad), constant_values=N_).reshape(IDXR, 128)
    attr2 = jnp.pad(attr, (0, pad), constant_values=1).reshape(IDXR, 128)
    g_idx, s_idx = _make_idx(row2, col2, attr2)

    X_lo, X_hi = _proj(x, Wv0, Wv1)

    z64 = jnp.zeros((ZROWS, HALF), jnp.float32)
    z16 = jnp.zeros((ZROWS, 16), jnp.float32)
    o16 = jnp.ones((128, 16), jnp.float32)

    S_lo, S_hi, cnt_col, cnt_row = _make_segsum(True)(
        X_lo, X_hi, g_idx, s_idx, z64, z16, o16)
    E_lo, E_hi = _eproj(S_lo, S_hi, cnt_col, We0, We1)
    R_lo, R_hi = _make_segsum(False)(E_lo, E_hi, s_idx, g_idx, z64)

    hn, o = _final(X_lo, X_hi, R_lo, R_hi, cnt_row,
                   Wu0, bu0.reshape(1, HID), Wu1, bu1.reshape(1, HID),
                   Wmix, bmix.reshape(1, HID), Wih, bih.reshape(1, 3 * HID),
                   bhh.reshape(1, 3 * HID), Wro, bro.reshape(1, HID))
    return hn, o[:, :3]


# speedup vs baseline: 7.3070x; 1.2970x over previous
import functools

import jax
import jax.numpy as jnp
from jax import lax
from jax.experimental import pallas as pl
from jax.experimental.pallas import tpu as pltpu
from jax.experimental.pallas import tpu_sc as plsc

N_ = 10000
E_ = 320000
HID = 128
HALF = 64
TWO_N = 2 * N_
TBL = 20480
TRASH = TWO_N
IDXR = 2560
EPAD = IDXR * 128
NTILES = 16
ROWS_PER_TILE = IDXR // NTILES
KROWS = 2
NCHUNK = ROWS_PER_TILE // KROWS
ZROWS = TBL // NTILES
CHUNK_E = KROWS * 128
CW = 8


def _idx_body(row_ref, col_ref, attr_ref, g_ref, s_ref):
    a = attr_ref[...]
    g_ref[...] = a * N_ + row_ref[...]
    s_ref[...] = a * N_ + col_ref[...]


def _make_idx(row2, col2, attr2):
    bs = pl.BlockSpec((256, 128), lambda i: (i, 0))
    return pl.pallas_call(
        _idx_body,
        grid=(10,),
        in_specs=[bs, bs, bs],
        out_specs=[bs, bs],
        out_shape=[jax.ShapeDtypeStruct((IDXR, 128), jnp.int32)] * 2,
    )(row2, col2, attr2)


def _proj_body(x_ref, w0_ref, w1_ref, lo_ref, hi_ref):
    i = pl.program_id(0)
    w = jnp.where(i < 10, w0_ref[...], w1_ref[...])
    p = jnp.dot(x_ref[...], w, preferred_element_type=jnp.float32)
    lo_ref[...] = p[:, :HALF]
    hi_ref[...] = p[:, HALF:]


def _proj(x, Wv0, Wv1):
    wspec = pl.BlockSpec((HID, HID), lambda i: (0, 0))
    ospec = pl.BlockSpec((1000, HALF), lambda i: (i, 0))
    return pl.pallas_call(
        _proj_body,
        grid=(20,),
        in_specs=[pl.BlockSpec((1000, HID), lambda i: (i % 10, 0)), wspec, wspec],
        out_specs=[ospec, ospec],
        out_shape=[jax.ShapeDtypeStruct((TBL, HALF), jnp.float32)] * 2,
    )(x, Wv0, Wv1)


def _seg_body(with_counts, *refs):
    if with_counts:
        (tlo, thi, gix, six, z64, zc, oc,
         out_lo, out_hi, cnt_a, cnt_b,
         g0, s0, g1, s1, r0, r1, ones_v, acc, cnt, sem0, sem1) = refs
    else:
        (tlo, thi, gix, six, z64,
         out_lo, out_hi,
         g0, s0, g1, s1, r0, r1, acc, sem0, sem1) = refs
        zc = oc = cnt_a = cnt_b = ones_v = cnt = None

    c = lax.axis_index("c")
    s = lax.axis_index("s")
    zsl = pl.ds(s * ZROWS, ZROWS)

    pltpu.sync_copy(z64, acc.at[zsl])
    if with_counts:
        pltpu.sync_copy(zc, cnt.at[zsl])
        pltpu.sync_copy(oc, ones_v)
    plsc.subcore_barrier()

    base = s * ROWS_PER_TILE
    gbufs, sbufs, rows, sems = (g0, g1), (s0, s1), (r0, r1), (sem0, sem1)

    def load_and_fire(row0, bi):
        pltpu.sync_copy(gix.at[pl.ds(row0, KROWS)], gbufs[bi])
        pltpu.sync_copy(six.at[pl.ds(row0, KROWS)], sbufs[bi])

        @pl.when(c == 0)
        def _():
            for j in range(KROWS):
                pltpu.async_copy(tlo.at[gbufs[bi].at[j]],
                                 rows[bi].at[pl.ds(j * 128, 128)], sems[bi])

        @pl.when(c == 1)
        def _():
            for j in range(KROWS):
                pltpu.async_copy(thi.at[gbufs[bi].at[j]],
                                 rows[bi].at[pl.ds(j * 128, 128)], sems[bi])

    def drain(bi):
        for j in range(KROWS):
            pltpu.make_async_copy(tlo.at[gbufs[bi].at[j]],
                                  rows[bi].at[pl.ds(j * 128, 128)],
                                  sems[bi]).wait()

    def scatter(bi):
        for j in range(KROWS):
            pltpu.sync_copy(rows[bi].at[pl.ds(j * 128, 128)],
                            acc.at[sbufs[bi].at[j]], add=True)
        if with_counts:
            @pl.when(c == 0)
            def _():
                for j in range(KROWS):
                    pltpu.sync_copy(ones_v, cnt.at[sbufs[bi].at[j]], add=True)

            @pl.when(c == 1)
            def _():
                for j in range(KROWS):
                    pltpu.sync_copy(ones_v, cnt.at[gbufs[bi].at[j]], add=True)

    npairs = NCHUNK // 2
    load_and_fire(base, 0)

    def pair(k, carry):
        load_and_fire(base + (2 * k + 1) * KROWS, 1)
        drain(0)
        scatter(0)

        @pl.when(k < npairs - 1)
        def _():
            load_and_fire(base + (2 * k + 2) * KROWS, 0)

        drain(1)
        scatter(1)
        return carry

    lax.fori_loop(0, npairs, pair, 0)
    plsc.subcore_barrier()

    @pl.when(c == 0)
    def _():
        pltpu.sync_copy(acc.at[zsl], out_lo.at[zsl])
        if with_counts:
            pltpu.sync_copy(cnt.at[zsl], cnt_a.at[zsl])

    @pl.when(c == 1)
    def _():
        pltpu.sync_copy(acc.at[zsl], out_hi.at[zsl])
        if with_counts:
            pltpu.sync_copy(cnt.at[zsl], cnt_b.at[zsl])


def _make_segsum(with_counts):
    mesh = plsc.VectorSubcoreMesh(core_axis_name="c", subcore_axis_name="s")
    out_type = [jax.ShapeDtypeStruct((TBL, HALF), jnp.float32)] * 2
    scratch = [
        pltpu.VMEM((KROWS, 128), jnp.int32),
        pltpu.VMEM((KROWS, 128), jnp.int32),
        pltpu.VMEM((KROWS, 128), jnp.int32),
        pltpu.VMEM((KROWS, 128), jnp.int32),
        pltpu.VMEM((CHUNK_E, HALF), jnp.float32),
        pltpu.VMEM((CHUNK_E, HALF), jnp.float32),
    ]
    if with_counts:
        out_type = out_type + [jax.ShapeDtypeStruct((TBL, CW), jnp.float32)] * 2
        scratch.append(pltpu.VMEM((128, CW), jnp.float32))
    scratch.append(pltpu.VMEM_SHARED((TBL, HALF), jnp.float32))
    if with_counts:
        scratch.append(pltpu.VMEM_SHARED((TBL, CW), jnp.float32))
    scratch.append(pltpu.SemaphoreType.DMA)
    scratch.append(pltpu.SemaphoreType.DMA)
    return pl.kernel(
        functools.partial(_seg_body, with_counts),
        out_type=out_type,
        mesh=mesh,
        scratch_types=scratch,
        compiler_params=pltpu.CompilerParams(use_tc_tiling_on_sc=False),
    )


def _eproj_body(sl_ref, sh_ref, cnt_ref, w0_ref, w1_ref, lo_ref, hi_ref):
    i = pl.program_id(0)
    cnt = jnp.maximum(cnt_ref[...][:, :1], 1.0)
    feat = jnp.concatenate([sl_ref[...], sh_ref[...]], axis=1) / cnt
    w = jnp.where(i < 10, w0_ref[...], w1_ref[...])
    p = jnp.dot(feat, w, preferred_element_type=jnp.float32)
    lo_ref[...] = p[:, :HALF]
    hi_ref[...] = p[:, HALF:]


def _eproj(S_lo, S_hi, cnt_col, We0, We1):
    hspec = pl.BlockSpec((1000, HALF), lambda i: (i, 0))
    wspec = pl.BlockSpec((HID, HID), lambda i: (0, 0))
    return pl.pallas_call(
        _eproj_body,
        grid=(20,),
        in_specs=[hspec, hspec, pl.BlockSpec((1000, CW), lambda i: (i, 0)),
                  wspec, wspec],
        out_specs=[hspec, hspec],
        out_shape=[jax.ShapeDtypeStruct((TBL, HALF), jnp.float32)] * 2,
    )(S_lo, S_hi, cnt_col, We0, We1)


def _final_body(xl0, xh0, xl1, xh1, rl0, rh0, rl1, rh1, c0, c1,
                wu0, bu0, wu1, bu1, wm, bm, wih, bih, bhh, wro, bro,
                hn_ref, o_ref):
    f32 = jnp.float32
    xp0 = jnp.concatenate([xl0[...], xh0[...]], axis=1)
    xp1 = jnp.concatenate([xl1[...], xh1[...]], axis=1)
    na0 = jnp.concatenate([rl0[...], rh0[...]], axis=1) / jnp.maximum(c0[...][:, :1], 1.0)
    na1 = jnp.concatenate([rl1[...], rh1[...]], axis=1) / jnp.maximum(c1[...][:, :1], 1.0)
    wu0v = wu0[...]
    wu1v = wu1[...]
    h0 = jax.nn.relu(jnp.dot(xp0, wu0v[:HID], preferred_element_type=f32)
                     + jnp.dot(na0, wu0v[HID:], preferred_element_type=f32)
                     + bu0[...])
    h1 = jax.nn.relu(jnp.dot(xp1, wu1v[:HID], preferred_element_type=f32)
                     + jnp.dot(na1, wu1v[HID:], preferred_element_type=f32)
                     + bu1[...])
    wmv = wm[...]
    h = jax.nn.relu(jnp.dot(h0, wmv[:HID], preferred_element_type=f32)
                    + jnp.dot(h1, wmv[HID:], preferred_element_type=f32)
                    + bm[...])
    gi = jnp.dot(h, wih[...], preferred_element_type=f32) + bih[...]
    bhhv = bhh[...]
    r = jax.nn.sigmoid(gi[:, :HID] + bhhv[:, :HID])
    z = jax.nn.sigmoid(gi[:, HID:2 * HID] + bhhv[:, HID:2 * HID])
    n = jnp.tanh(gi[:, 2 * HID:] + r * bhhv[:, 2 * HID:])
    hn = (1.0 - z) * n
    hn_ref[...] = hn
    o_ref[...] = jnp.dot(hn, wro[...], preferred_element_type=f32) + bro[...]


def _final(X_lo, X_hi, R_lo, R_hi, cnt_row,
           Wu0, bu0, Wu1, bu1, Wmix, bmix, Wih, bih, bhh, Wro, bro):
    def hblk(off):
        return pl.BlockSpec((1000, HALF), lambda i, o=off: (i + o, 0))

    def cblk(off):
        return pl.BlockSpec((1000, CW), lambda i, o=off: (i + o, 0))

    def whole(a):
        return pl.BlockSpec(a.shape, lambda i: tuple(0 for _ in a.shape))

    ospec = pl.BlockSpec((1000, HID), lambda i: (i, 0))
    return pl.pallas_call(
        _final_body,
        grid=(10,),
        in_specs=[hblk(0), hblk(0), hblk(10), hblk(10),
                  hblk(0), hblk(0), hblk(10), hblk(10),
                  cblk(0), cblk(10),
                  whole(Wu0), whole(bu0), whole(Wu1), whole(bu1),
                  whole(Wmix), whole(bmix), whole(Wih), whole(bih),
                  whole(bhh), whole(Wro), whole(bro)],
        out_specs=[ospec, ospec],
        out_shape=[jax.ShapeDtypeStruct((N_, HID), jnp.float32)] * 2,
    )(X_lo, X_hi, X_lo, X_hi, R_lo, R_hi, R_lo, R_hi, cnt_row, cnt_row,
      Wu0, bu0, Wu1, bu1, Wmix, bmix, Wih, bih, bhh, Wro, bro)


def kernel(x, edge_index, edge_attr, Wv0, We0, Wu0, bu0, Wv1, We1, Wu1, bu1,
           Wmix, bmix, Wih, Whh, bih, bhh, Wro, bro):
    del Whh
    row = edge_index[0].astype(jnp.int32)
    col = edge_index[1].astype(jnp.int32)
    attr = edge_attr.astype(jnp.int32)
    pad = EPAD - E_
    row2 = jnp.pad(row, (0, pad), constant_values=N_).reshape(IDXR, 128)
    col2 = jnp.pad(col, (0, pad), constant_values=N_).reshape(IDXR, 128)
    attr2 = jnp.pad(attr, (0, pad), constant_values=1).reshape(IDXR, 128)
    g_idx, s_idx = _make_idx(row2, col2, attr2)

    X_lo, X_hi = _proj(x, Wv0, Wv1)

    z64 = jnp.zeros((ZROWS, HALF), jnp.float32)
    zc = jnp.zeros((ZROWS, CW), jnp.float32)
    oc = jnp.ones((128, CW), jnp.float32)

    S_lo, S_hi, cnt_col, cnt_row = _make_segsum(True)(
        X_lo, X_hi, g_idx, s_idx, z64, zc, oc)
    E_lo, E_hi = _eproj(S_lo, S_hi, cnt_col, We0, We1)
    R_lo, R_hi = _make_segsum(False)(E_lo, E_hi, s_idx, g_idx, z64)

    hn, o = _final(X_lo, X_hi, R_lo, R_hi, cnt_row,
                   Wu0, bu0.reshape(1, HID), Wu1, bu1.reshape(1, HID),
                   Wmix, bmix.reshape(1, HID), Wih, bih.reshape(1, 3 * HID),
                   bhh.reshape(1, 3 * HID), Wro, bro.reshape(1, HID))
    return hn, o[:, :3]


# speedup vs baseline: 7.6455x; 1.0463x over previous
import functools

import jax
import jax.numpy as jnp
from jax import lax
from jax.experimental import pallas as pl
from jax.experimental.pallas import tpu as pltpu
from jax.experimental.pallas import tpu_sc as plsc

N_ = 10000
E_ = 320000
HID = 128
HALF = 64
TWO_N = 2 * N_
TBL = 20480
TRASH = TWO_N
IDXR = 2560
EPAD = IDXR * 128
NTILES = 16
ROWS_PER_TILE = IDXR // NTILES
SUPER = 16
ZROWS = TBL // NTILES
CW = 8


def _idx_body(row_ref, col_ref, attr_ref, g_ref, s_ref):
    a = attr_ref[...]
    g_ref[...] = a * N_ + row_ref[...]
    s_ref[...] = a * N_ + col_ref[...]


def _make_idx(row2, col2, attr2):
    bs = pl.BlockSpec((256, 128), lambda i: (i, 0))
    return pl.pallas_call(
        _idx_body,
        grid=(10,),
        in_specs=[bs, bs, bs],
        out_specs=[bs, bs],
        out_shape=[jax.ShapeDtypeStruct((IDXR, 128), jnp.int32)] * 2,
    )(row2, col2, attr2)


def _proj_body(x_ref, w0_ref, w1_ref, lo_ref, hi_ref):
    i = pl.program_id(0)
    w = jnp.where(i < 10, w0_ref[...], w1_ref[...])
    p = jnp.dot(x_ref[...], w, preferred_element_type=jnp.float32)
    lo_ref[...] = p[:, :HALF]
    hi_ref[...] = p[:, HALF:]


def _proj(x, Wv0, Wv1):
    wspec = pl.BlockSpec((HID, HID), lambda i: (0, 0))
    ospec = pl.BlockSpec((1000, HALF), lambda i: (i, 0))
    return pl.pallas_call(
        _proj_body,
        grid=(20,),
        in_specs=[pl.BlockSpec((1000, HID), lambda i: (i % 10, 0)), wspec, wspec],
        out_specs=[ospec, ospec],
        out_shape=[jax.ShapeDtypeStruct((TBL, HALF), jnp.float32)] * 2,
    )(x, Wv0, Wv1)


def _seg_body(with_counts, *refs):
    if with_counts:
        (tlo, thi, gix, six, z64, zc, oc,
         out_lo, out_hi, cnt_a, cnt_b,
         gbuf, sbuf, rb0, rb1, rb2, ones_v, acc, cnt,
         sg0, sg1, sg2, ss0, ss1, ss2, semc) = refs
    else:
        (tlo, thi, gix, six, z64,
         out_lo, out_hi,
         gbuf, sbuf, rb0, rb1, rb2, acc,
         sg0, sg1, sg2, ss0, ss1, ss2) = refs
        zc = oc = cnt_a = cnt_b = ones_v = cnt = semc = None

    c = lax.axis_index("c")
    s = lax.axis_index("s")
    zsl = pl.ds(s * ZROWS, ZROWS)

    pltpu.sync_copy(z64, acc.at[zsl])
    if with_counts:
        pltpu.sync_copy(zc, cnt.at[zsl])
        pltpu.sync_copy(oc, ones_v)
    plsc.subcore_barrier()

    base = s * ROWS_PER_TILE
    rbufs = (rb0, rb1, rb2)
    semg = (sg0, sg1, sg2)
    sems = (ss0, ss1, ss2)

    def fire_g(j):
        b = j % 3

        @pl.when(c == 0)
        def _():
            pltpu.async_copy(tlo.at[gbuf.at[j]], rbufs[b], semg[b])

        @pl.when(c == 1)
        def _():
            pltpu.async_copy(thi.at[gbuf.at[j]], rbufs[b], semg[b])

    def super_body(i, carry):
        r0 = base + i * SUPER
        pltpu.sync_copy(gix.at[pl.ds(r0, SUPER)], gbuf)
        pltpu.sync_copy(six.at[pl.ds(r0, SUPER)], sbuf)
        fire_g(0)
        fire_g(1)
        for j in range(SUPER):
            b = j % 3
            pltpu.make_async_copy(tlo.at[gbuf.at[j]], rbufs[b], semg[b]).wait()
            pltpu.async_copy(rbufs[b], acc.at[sbuf.at[j]], sems[b], add=True)
            if with_counts:
                @pl.when(c == 0)
                def _():
                    pltpu.async_copy(ones_v, cnt.at[sbuf.at[j]], semc, add=True)

                @pl.when(c == 1)
                def _():
                    pltpu.async_copy(ones_v, cnt.at[gbuf.at[j]], semc, add=True)
            if j + 2 < SUPER:
                if j >= 1:
                    nb = (j + 2) % 3
                    pltpu.make_async_copy(rbufs[nb], acc.at[sbuf.at[j - 1]],
                                          sems[nb]).wait()
                fire_g(j + 2)
        for j in (SUPER - 3, SUPER - 2, SUPER - 1):
            pltpu.make_async_copy(rbufs[j % 3], acc.at[sbuf.at[j]],
                                  sems[j % 3]).wait()
        if with_counts:
            for j in range(SUPER):
                pltpu.make_async_copy(ones_v, cnt.at[sbuf.at[j]], semc).wait()
        return carry

    lax.fori_loop(0, ROWS_PER_TILE // SUPER, super_body, 0)
    plsc.subcore_barrier()

    @pl.when(c == 0)
    def _():
        pltpu.sync_copy(acc.at[zsl], out_lo.at[zsl])
        if with_counts:
            pltpu.sync_copy(cnt.at[zsl], cnt_a.at[zsl])

    @pl.when(c == 1)
    def _():
        pltpu.sync_copy(acc.at[zsl], out_hi.at[zsl])
        if with_counts:
            pltpu.sync_copy(cnt.at[zsl], cnt_b.at[zsl])


def _make_segsum(with_counts):
    mesh = plsc.VectorSubcoreMesh(core_axis_name="c", subcore_axis_name="s")
    out_type = [jax.ShapeDtypeStruct((TBL, HALF), jnp.float32)] * 2
    scratch = [
        pltpu.VMEM((SUPER, 128), jnp.int32),
        pltpu.VMEM((SUPER, 128), jnp.int32),
        pltpu.VMEM((128, HALF), jnp.float32),
        pltpu.VMEM((128, HALF), jnp.float32),
        pltpu.VMEM((128, HALF), jnp.float32),
    ]
    if with_counts:
        out_type = out_type + [jax.ShapeDtypeStruct((TBL, CW), jnp.float32)] * 2
        scratch.append(pltpu.VMEM((128, CW), jnp.float32))
    scratch.append(pltpu.VMEM_SHARED((TBL, HALF), jnp.float32))
    if with_counts:
        scratch.append(pltpu.VMEM_SHARED((TBL, CW), jnp.float32))
    scratch.extend([pltpu.SemaphoreType.DMA] * 6)
    if with_counts:
        scratch.append(pltpu.SemaphoreType.DMA)
    return pl.kernel(
        functools.partial(_seg_body, with_counts),
        out_type=out_type,
        mesh=mesh,
        scratch_types=scratch,
        compiler_params=pltpu.CompilerParams(use_tc_tiling_on_sc=False),
    )


def _eproj_body(sl_ref, sh_ref, cnt_ref, w0_ref, w1_ref, lo_ref, hi_ref):
    i = pl.program_id(0)
    cnt = jnp.maximum(cnt_ref[...][:, :1], 1.0)
    feat = jnp.concatenate([sl_ref[...], sh_ref[...]], axis=1) / cnt
    w = jnp.where(i < 10, w0_ref[...], w1_ref[...])
    p = jnp.dot(feat, w, preferred_element_type=jnp.float32)
    lo_ref[...] = p[:, :HALF]
    hi_ref[...] = p[:, HALF:]


def _eproj(S_lo, S_hi, cnt_col, We0, We1):
    hspec = pl.BlockSpec((1000, HALF), lambda i: (i, 0))
    wspec = pl.BlockSpec((HID, HID), lambda i: (0, 0))
    return pl.pallas_call(
        _eproj_body,
        grid=(20,),
        in_specs=[hspec, hspec, pl.BlockSpec((1000, CW), lambda i: (i, 0)),
                  wspec, wspec],
        out_specs=[hspec, hspec],
        out_shape=[jax.ShapeDtypeStruct((TBL, HALF), jnp.float32)] * 2,
    )(S_lo, S_hi, cnt_col, We0, We1)


def _final_body(xl0, xh0, xl1, xh1, rl0, rh0, rl1, rh1, c0, c1,
                wu0, bu0, wu1, bu1, wm, bm, wih, bih, bhh, wro, bro,
                hn_ref, o_ref):
    f32 = jnp.float32
    xp0 = jnp.concatenate([xl0[...], xh0[...]], axis=1)
    xp1 = jnp.concatenate([xl1[...], xh1[...]], axis=1)
    na0 = jnp.concatenate([rl0[...], rh0[...]], axis=1) / jnp.maximum(c0[...][:, :1], 1.0)
    na1 = jnp.concatenate([rl1[...], rh1[...]], axis=1) / jnp.maximum(c1[...][:, :1], 1.0)
    wu0v = wu0[...]
    wu1v = wu1[...]
    h0 = jax.nn.relu(jnp.dot(xp0, wu0v[:HID], preferred_element_type=f32)
                     + jnp.dot(na0, wu0v[HID:], preferred_element_type=f32)
                     + bu0[...])
    h1 = jax.nn.relu(jnp.dot(xp1, wu1v[:HID], preferred_element_type=f32)
                     + jnp.dot(na1, wu1v[HID:], preferred_element_type=f32)
                     + bu1[...])
    wmv = wm[...]
    h = jax.nn.relu(jnp.dot(h0, wmv[:HID], preferred_element_type=f32)
                    + jnp.dot(h1, wmv[HID:], preferred_element_type=f32)
                    + bm[...])
    gi = jnp.dot(h, wih[...], preferred_element_type=f32) + bih[...]
    bhhv = bhh[...]
    r = jax.nn.sigmoid(gi[:, :HID] + bhhv[:, :HID])
    z = jax.nn.sigmoid(gi[:, HID:2 * HID] + bhhv[:, HID:2 * HID])
    n = jnp.tanh(gi[:, 2 * HID:] + r * bhhv[:, 2 * HID:])
    hn = (1.0 - z) * n
    hn_ref[...] = hn
    o_ref[...] = jnp.dot(hn, wro[...], preferred_element_type=f32) + bro[...]


def _final(X_lo, X_hi, R_lo, R_hi, cnt_row,
           Wu0, bu0, Wu1, bu1, Wmix, bmix, Wih, bih, bhh, Wro, bro):
    def hblk(off):
        return pl.BlockSpec((1000, HALF), lambda i, o=off: (i + o, 0))

    def cblk(off):
        return pl.BlockSpec((1000, CW), lambda i, o=off: (i + o, 0))

    def whole(a):
        return pl.BlockSpec(a.shape, lambda i: tuple(0 for _ in a.shape))

    ospec = pl.BlockSpec((1000, HID), lambda i: (i, 0))
    return pl.pallas_call(
        _final_body,
        grid=(10,),
        in_specs=[hblk(0), hblk(0), hblk(10), hblk(10),
                  hblk(0), hblk(0), hblk(10), hblk(10),
                  cblk(0), cblk(10),
                  whole(Wu0), whole(bu0), whole(Wu1), whole(bu1),
                  whole(Wmix), whole(bmix), whole(Wih), whole(bih),
                  whole(bhh), whole(Wro), whole(bro)],
        out_specs=[ospec, ospec],
        out_shape=[jax.ShapeDtypeStruct((N_, HID), jnp.float32)] * 2,
    )(X_lo, X_hi, X_lo, X_hi, R_lo, R_hi, R_lo, R_hi, cnt_row, cnt_row,
      Wu0, bu0, Wu1, bu1, Wmix, bmix, Wih, bih, bhh, Wro, bro)


def kernel(x, edge_index, edge_attr, Wv0, We0, Wu0, bu0, Wv1, We1, Wu1, bu1,
           Wmix, bmix, Wih, Whh, bih, bhh, Wro, bro):
    del Whh
    row = edge_index[0].astype(jnp.int32)
    col = edge_index[1].astype(jnp.int32)
    attr = edge_attr.astype(jnp.int32)
    pad = EPAD - E_
    row2 = jnp.pad(row, (0, pad), constant_values=N_).reshape(IDXR, 128)
    col2 = jnp.pad(col, (0, pad), constant_values=N_).reshape(IDXR, 128)
    attr2 = jnp.pad(attr, (0, pad), constant_values=1).reshape(IDXR, 128)
    g_idx, s_idx = _make_idx(row2, col2, attr2)

    X_lo, X_hi = _proj(x, Wv0, Wv1)

    z64 = jnp.zeros((ZROWS, HALF), jnp.float32)
    zc = jnp.zeros((ZROWS, CW), jnp.float32)
    oc = jnp.ones((128, CW), jnp.float32)

    S_lo, S_hi, cnt_col, cnt_row = _make_segsum(True)(
        X_lo, X_hi, g_idx, s_idx, z64, zc, oc)
    E_lo, E_hi = _eproj(S_lo, S_hi, cnt_col, We0, We1)
    R_lo, R_hi = _make_segsum(False)(E_lo, E_hi, s_idx, g_idx, z64)

    hn, o = _final(X_lo, X_hi, R_lo, R_hi, cnt_row,
                   Wu0, bu0.reshape(1, HID), Wu1, bu1.reshape(1, HID),
                   Wmix, bmix.reshape(1, HID), Wih, bih.reshape(1, 3 * HID),
                   bhh.reshape(1, 3 * HID), Wro, bro.reshape(1, HID))
    return hn, o[:, :3]


# speedup vs baseline: 7.7039x; 1.0076x over previous
import functools

import jax
import jax.numpy as jnp
from jax import lax
from jax.experimental import pallas as pl
from jax.experimental.pallas import tpu as pltpu
from jax.experimental.pallas import tpu_sc as plsc

N_ = 10000
E_ = 320000
HID = 128
HALF = 64
TWO_N = 2 * N_
TBL = 20480
TRASH = TWO_N
IDXR = 2560
EPAD = IDXR * 128
NTILES = 16
ROWS_PER_TILE = IDXR // NTILES
SUPER = 16
ZROWS = TBL // NTILES
CW = 8


def _proj_body(x_ref, w0_ref, w1_ref, row_ref, col_ref, attr_ref,
               lo_ref, hi_ref, g_ref, s_ref):
    i = pl.program_id(0)
    w = jnp.where(i < 10, w0_ref[...], w1_ref[...])
    p = jnp.dot(x_ref[...], w, preferred_element_type=jnp.float32)
    lo_ref[...] = p[:, :HALF]
    hi_ref[...] = p[:, HALF:]
    a = attr_ref[...]
    g_ref[...] = a * N_ + row_ref[...]
    s_ref[...] = a * N_ + col_ref[...]


def _proj(x, Wv0, Wv1, row2, col2, attr2):
    wspec = pl.BlockSpec((HID, HID), lambda i: (0, 0))
    ospec = pl.BlockSpec((1000, HALF), lambda i: (i, 0))
    ispec = pl.BlockSpec((128, 128), lambda i: (i, 0))
    return pl.pallas_call(
        _proj_body,
        grid=(20,),
        in_specs=[pl.BlockSpec((1000, HID), lambda i: (i % 10, 0)), wspec, wspec,
                  ispec, ispec, ispec],
        out_specs=[ospec, ospec, ispec, ispec],
        out_shape=[jax.ShapeDtypeStruct((TBL, HALF), jnp.float32)] * 2
        + [jax.ShapeDtypeStruct((IDXR, 128), jnp.int32)] * 2,
    )(x, Wv0, Wv1, row2, col2, attr2)


def _seg_body(with_counts, *refs):
    if with_counts:
        (tlo, thi, gix, six, z64, zc, oc,
         out_lo, out_hi, cnt_a, cnt_b,
         gbuf, sbuf, rb0, rb1, rb2, ones_v, acc, cnt,
         sg0, sg1, sg2, ss0, ss1, ss2, semc) = refs
    else:
        (tlo, thi, gix, six, z64,
         out_lo, out_hi,
         gbuf, sbuf, rb0, rb1, rb2, acc,
         sg0, sg1, sg2, ss0, ss1, ss2) = refs
        zc = oc = cnt_a = cnt_b = ones_v = cnt = semc = None

    c = lax.axis_index("c")
    s = lax.axis_index("s")
    zsl = pl.ds(s * ZROWS, ZROWS)

    pltpu.sync_copy(z64, acc.at[zsl])
    if with_counts:
        pltpu.sync_copy(zc, cnt.at[zsl])
        pltpu.sync_copy(oc, ones_v)
    plsc.subcore_barrier()

    base = s * ROWS_PER_TILE
    rbufs = (rb0, rb1, rb2)
    semg = (sg0, sg1, sg2)
    sems = (ss0, ss1, ss2)

    def fire_g(j):
        b = j % 3

        @pl.when(c == 0)
        def _():
            pltpu.async_copy(tlo.at[gbuf.at[j]], rbufs[b], semg[b])

        @pl.when(c == 1)
        def _():
            pltpu.async_copy(thi.at[gbuf.at[j]], rbufs[b], semg[b])

    def super_body(i, carry):
        r0 = base + i * SUPER
        pltpu.sync_copy(gix.at[pl.ds(r0, SUPER)], gbuf)
        pltpu.sync_copy(six.at[pl.ds(r0, SUPER)], sbuf)
        fire_g(0)
        fire_g(1)
        for j in range(SUPER):
            b = j % 3
            pltpu.make_async_copy(tlo.at[gbuf.at[j]], rbufs[b], semg[b]).wait()
            pltpu.async_copy(rbufs[b], acc.at[sbuf.at[j]], sems[b], add=True)
            if with_counts:
                @pl.when(c == 0)
                def _():
                    pltpu.async_copy(ones_v, cnt.at[sbuf.at[j]], semc, add=True)

                @pl.when(c == 1)
                def _():
                    pltpu.async_copy(ones_v, cnt.at[gbuf.at[j]], semc, add=True)
            if j + 2 < SUPER:
                if j >= 1:
                    nb = (j + 2) % 3
                    pltpu.make_async_copy(rbufs[nb], acc.at[sbuf.at[j - 1]],
                                          sems[nb]).wait()
                fire_g(j + 2)
        for j in (SUPER - 3, SUPER - 2, SUPER - 1):
            pltpu.make_async_copy(rbufs[j % 3], acc.at[sbuf.at[j]],
                                  sems[j % 3]).wait()
        if with_counts:
            for j in range(SUPER):
                pltpu.make_async_copy(ones_v, cnt.at[sbuf.at[j]], semc).wait()
        return carry

    lax.fori_loop(0, ROWS_PER_TILE // SUPER, super_body, 0)
    plsc.subcore_barrier()

    @pl.when(c == 0)
    def _():
        pltpu.sync_copy(acc.at[zsl], out_lo.at[zsl])
        if with_counts:
            pltpu.sync_copy(cnt.at[zsl], cnt_a.at[zsl])

    @pl.when(c == 1)
    def _():
        pltpu.sync_copy(acc.at[zsl], out_hi.at[zsl])
        if with_counts:
            pltpu.sync_copy(cnt.at[zsl], cnt_b.at[zsl])


def _make_segsum(with_counts):
    mesh = plsc.VectorSubcoreMesh(core_axis_name="c", subcore_axis_name="s")
    out_type = [jax.ShapeDtypeStruct((TBL, HALF), jnp.float32)] * 2
    scratch = [
        pltpu.VMEM((SUPER, 128), jnp.int32),
        pltpu.VMEM((SUPER, 128), jnp.int32),
        pltpu.VMEM((128, HALF), jnp.float32),
        pltpu.VMEM((128, HALF), jnp.float32),
        pltpu.VMEM((128, HALF), jnp.float32),
    ]
    if with_counts:
        out_type = out_type + [jax.ShapeDtypeStruct((TBL, CW), jnp.float32)] * 2
        scratch.append(pltpu.VMEM((128, CW), jnp.float32))
    scratch.append(pltpu.VMEM_SHARED((TBL, HALF), jnp.float32))
    if with_counts:
        scratch.append(pltpu.VMEM_SHARED((TBL, CW), jnp.float32))
    scratch.extend([pltpu.SemaphoreType.DMA] * 6)
    if with_counts:
        scratch.append(pltpu.SemaphoreType.DMA)
    return pl.kernel(
        functools.partial(_seg_body, with_counts),
        out_type=out_type,
        mesh=mesh,
        scratch_types=scratch,
        compiler_params=pltpu.CompilerParams(use_tc_tiling_on_sc=False),
    )


def _norm_body(sl_ref, sh_ref, cnt_ref, lo_ref, hi_ref):
    cnt = jnp.maximum(cnt_ref[...][:, :1], 1.0)
    lo_ref[...] = sl_ref[...] / cnt
    hi_ref[...] = sh_ref[...] / cnt


def _norm(S_lo, S_hi, cnt_col):
    hspec = pl.BlockSpec((1000, HALF), lambda i: (i, 0))
    return pl.pallas_call(
        _norm_body,
        grid=(20,),
        in_specs=[hspec, hspec, pl.BlockSpec((1000, CW), lambda i: (i, 0))],
        out_specs=[hspec, hspec],
        out_shape=[jax.ShapeDtypeStruct((TBL, HALF), jnp.float32)] * 2,
    )(S_lo, S_hi, cnt_col)


def _final_body(xl0, xh0, xl1, xh1, rl0, rh0, rl1, rh1, c0, c1,
                we0, we1, wu0, bu0, wu1, bu1, wm, bm, wih, bih, bhh, wro, bro,
                hn_ref, o_ref):
    f32 = jnp.float32
    xp0 = jnp.concatenate([xl0[...], xh0[...]], axis=1)
    xp1 = jnp.concatenate([xl1[...], xh1[...]], axis=1)
    na0 = jnp.concatenate([rl0[...], rh0[...]], axis=1) / jnp.maximum(c0[...][:, :1], 1.0)
    na1 = jnp.concatenate([rl1[...], rh1[...]], axis=1) / jnp.maximum(c1[...][:, :1], 1.0)
    na0 = jnp.dot(na0, we0[...], preferred_element_type=f32)
    na1 = jnp.dot(na1, we1[...], preferred_element_type=f32)
    wu0v = wu0[...]
    wu1v = wu1[...]
    h0 = jax.nn.relu(jnp.dot(xp0, wu0v[:HID], preferred_element_type=f32)
                     + jnp.dot(na0, wu0v[HID:], preferred_element_type=f32)
                     + bu0[...])
    h1 = jax.nn.relu(jnp.dot(xp1, wu1v[:HID], preferred_element_type=f32)
                     + jnp.dot(na1, wu1v[HID:], preferred_element_type=f32)
                     + bu1[...])
    wmv = wm[...]
    h = jax.nn.relu(jnp.dot(h0, wmv[:HID], preferred_element_type=f32)
                    + jnp.dot(h1, wmv[HID:], preferred_element_type=f32)
                    + bm[...])
    gi = jnp.dot(h, wih[...], preferred_element_type=f32) + bih[...]
    bhhv = bhh[...]
    r = jax.nn.sigmoid(gi[:, :HID] + bhhv[:, :HID])
    z = jax.nn.sigmoid(gi[:, HID:2 * HID] + bhhv[:, HID:2 * HID])
    n = jnp.tanh(gi[:, 2 * HID:] + r * bhhv[:, 2 * HID:])
    hn = (1.0 - z) * n
    hn_ref[...] = hn
    o_ref[...] = jnp.dot(hn, wro[...], preferred_element_type=f32) + bro[...]


def _final(X_lo, X_hi, R_lo, R_hi, cnt_row,
           We0, We1, Wu0, bu0, Wu1, bu1, Wmix, bmix, Wih, bih, bhh, Wro, bro):
    def hblk(off):
        return pl.BlockSpec((1000, HALF), lambda i, o=off: (i + o, 0))

    def cblk(off):
        return pl.BlockSpec((1000, CW), lambda i, o=off: (i + o, 0))

    def whole(a):
        return pl.BlockSpec(a.shape, lambda i: tuple(0 for _ in a.shape))

    ospec = pl.BlockSpec((1000, HID), lambda i: (i, 0))
    return pl.pallas_call(
        _final_body,
        grid=(10,),
        in_specs=[hblk(0), hblk(0), hblk(10), hblk(10),
                  hblk(0), hblk(0), hblk(10), hblk(10),
                  cblk(0), cblk(10),
                  whole(We0), whole(We1),
                  whole(Wu0), whole(bu0), whole(Wu1), whole(bu1),
                  whole(Wmix), whole(bmix), whole(Wih), whole(bih),
                  whole(bhh), whole(Wro), whole(bro)],
        out_specs=[ospec, ospec],
        out_shape=[jax.ShapeDtypeStruct((N_, HID), jnp.float32)] * 2,
    )(X_lo, X_hi, X_lo, X_hi, R_lo, R_hi, R_lo, R_hi, cnt_row, cnt_row,
      We0, We1, Wu0, bu0, Wu1, bu1, Wmix, bmix, Wih, bih, bhh, Wro, bro)


def kernel(x, edge_index, edge_attr, Wv0, We0, Wu0, bu0, Wv1, We1, Wu1, bu1,
           Wmix, bmix, Wih, Whh, bih, bhh, Wro, bro):
    del Whh
    row = edge_index[0].astype(jnp.int32)
    col = edge_index[1].astype(jnp.int32)
    attr = edge_attr.astype(jnp.int32)
    pad = EPAD - E_
    row2 = jnp.pad(row, (0, pad), constant_values=N_).reshape(IDXR, 128)
    col2 = jnp.pad(col, (0, pad), constant_values=N_).reshape(IDXR, 128)
    attr2 = jnp.pad(attr, (0, pad), constant_values=1).reshape(IDXR, 128)

    X_lo, X_hi, g_idx, s_idx = _proj(x, Wv0, Wv1, row2, col2, attr2)

    z64 = jnp.zeros((ZROWS, HALF), jnp.float32)
    zc = jnp.zeros((ZROWS, CW), jnp.float32)
    oc = jnp.ones((128, CW), jnp.float32)

    S_lo, S_hi, cnt_col, cnt_row = _make_segsum(True)(
        X_lo, X_hi, g_idx, s_idx, z64, zc, oc)
    F_lo, F_hi = _norm(S_lo, S_hi, cnt_col)
    R_lo, R_hi = _make_segsum(False)(F_lo, F_hi, s_idx, g_idx, z64)

    hn, o = _final(X_lo, X_hi, R_lo, R_hi, cnt_row, We0, We1,
                   Wu0, bu0.reshape(1, HID), Wu1, bu1.reshape(1, HID),
                   Wmix, bmix.reshape(1, HID), Wih, bih.reshape(1, 3 * HID),
                   bhh.reshape(1, 3 * HID), Wro, bro.reshape(1, HID))
    return hn, o[:, :3]


# speedup vs baseline: 7.8633x; 1.0207x over previous
import functools

import jax
import jax.numpy as jnp
from jax import lax
from jax.experimental import pallas as pl
from jax.experimental.pallas import tpu as pltpu
from jax.experimental.pallas import tpu_sc as plsc

N_ = 10000
E_ = 320000
HID = 128
HALF = 64
TWO_N = 2 * N_
TBL = 20480
TRASH = TWO_N
IDXR = 2560
EPAD = IDXR * 128
NTILES = 16
ROWS_PER_TILE = IDXR // NTILES
SUPER = 16
ZROWS = TBL // NTILES
CW = 8


def _proj_body(x_ref, w0_ref, w1_ref, row_ref, col_ref, attr_ref,
               lo_ref, hi_ref, g_ref, s_ref):
    i = pl.program_id(0)
    w = jnp.where(i < 10, w0_ref[...], w1_ref[...])
    p = jnp.dot(x_ref[...], w, preferred_element_type=jnp.float32)
    lo_ref[...] = p[:, :HALF]
    hi_ref[...] = p[:, HALF:]
    a = attr_ref[...]
    g_ref[...] = a * N_ + row_ref[...]
    s_ref[...] = a * N_ + col_ref[...]


def _proj(x, Wv0, Wv1, row2, col2, attr2):
    wspec = pl.BlockSpec((HID, HID), lambda i: (0, 0))
    ospec = pl.BlockSpec((1000, HALF), lambda i: (i, 0))
    ispec = pl.BlockSpec((128, 128), lambda i: (i, 0))
    return pl.pallas_call(
        _proj_body,
        grid=(20,),
        in_specs=[pl.BlockSpec((1000, HID), lambda i: (i % 10, 0)), wspec, wspec,
                  ispec, ispec, ispec],
        out_specs=[ospec, ospec, ispec, ispec],
        out_shape=[jax.ShapeDtypeStruct((TBL, HALF), jnp.float32)] * 2
        + [jax.ShapeDtypeStruct((IDXR, 128), jnp.int32)] * 2,
    )(x, Wv0, Wv1, row2, col2, attr2)


def _seg_body(with_counts, *refs):
    if with_counts:
        (tlo, thi, gix, six, z64, zc, oc,
         out_lo, out_hi, cnt_a, cnt_b,
         gb0, sb0, gb1, sb1, rb0, rb1, rb2, ones_v, acc, cnt,
         sg0, sg1, sg2, ss0, ss1, ss2, semc, si0, si1) = refs
    else:
        (tlo, thi, gix, six, z64,
         out_lo, out_hi,
         gb0, sb0, gb1, sb1, rb0, rb1, rb2, acc,
         sg0, sg1, sg2, ss0, ss1, ss2, si0, si1) = refs
        zc = oc = cnt_a = cnt_b = ones_v = cnt = semc = None

    c = lax.axis_index("c")
    s = lax.axis_index("s")
    zsl = pl.ds(s * ZROWS, ZROWS)

    pltpu.sync_copy(z64, acc.at[zsl])
    if with_counts:
        pltpu.sync_copy(zc, cnt.at[zsl])
        pltpu.sync_copy(oc, ones_v)
    plsc.subcore_barrier()

    base = s * ROWS_PER_TILE
    rbufs = (rb0, rb1, rb2)
    semg = (sg0, sg1, sg2)
    sems = (ss0, ss1, ss2)
    gbufs, sbufs, semi = (gb0, gb1), (sb0, sb1), (si0, si1)

    def fire_idx(r0, bi):
        pltpu.async_copy(gix.at[pl.ds(r0, SUPER)], gbufs[bi], semi[bi])
        pltpu.async_copy(six.at[pl.ds(r0, SUPER)], sbufs[bi], semi[bi])

    def wait_idx(bi):
        pltpu.make_async_copy(gix.at[pl.ds(0, SUPER)], gbufs[bi],
                              semi[bi]).wait()
        pltpu.make_async_copy(six.at[pl.ds(0, SUPER)], sbufs[bi],
                              semi[bi]).wait()

    def process_super(bi):
        gbuf, sbuf = gbufs[bi], sbufs[bi]

        def fire_g(j):
            b = j % 3

            @pl.when(c == 0)
            def _():
                pltpu.async_copy(tlo.at[gbuf.at[j]], rbufs[b], semg[b])

            @pl.when(c == 1)
            def _():
                pltpu.async_copy(thi.at[gbuf.at[j]], rbufs[b], semg[b])

        fire_g(0)
        fire_g(1)
        for j in range(SUPER):
            b = j % 3
            pltpu.make_async_copy(tlo.at[gbuf.at[j]], rbufs[b], semg[b]).wait()
            pltpu.async_copy(rbufs[b], acc.at[sbuf.at[j]], sems[b], add=True)
            if with_counts:
                @pl.when(c == 0)
                def _():
                    pltpu.async_copy(ones_v, cnt.at[sbuf.at[j]], semc, add=True)

                @pl.when(c == 1)
                def _():
                    pltpu.async_copy(ones_v, cnt.at[gbuf.at[j]], semc, add=True)
            if j + 2 < SUPER:
                if j >= 1:
                    nb = (j + 2) % 3
                    pltpu.make_async_copy(rbufs[nb], acc.at[sbuf.at[j - 1]],
                                          sems[nb]).wait()
                fire_g(j + 2)
        for j in (SUPER - 3, SUPER - 2, SUPER - 1):
            pltpu.make_async_copy(rbufs[j % 3], acc.at[sbuf.at[j]],
                                  sems[j % 3]).wait()
        if with_counts:
            for j in range(SUPER):
                pltpu.make_async_copy(ones_v, cnt.at[sbuf.at[j]], semc).wait()

    npairs = ROWS_PER_TILE // SUPER // 2
    fire_idx(base, 0)

    def pair(k, carry):
        fire_idx(base + (2 * k + 1) * SUPER, 1)
        wait_idx(0)
        process_super(0)

        @pl.when(k < npairs - 1)
        def _():
            fire_idx(base + (2 * k + 2) * SUPER, 0)

        wait_idx(1)
        process_super(1)
        return carry

    lax.fori_loop(0, npairs, pair, 0)
    plsc.subcore_barrier()

    @pl.when(c == 0)
    def _():
        pltpu.sync_copy(acc.at[zsl], out_lo.at[zsl])
        if with_counts:
            pltpu.sync_copy(cnt.at[zsl], cnt_a.at[zsl])

    @pl.when(c == 1)
    def _():
        pltpu.sync_copy(acc.at[zsl], out_hi.at[zsl])
        if with_counts:
            pltpu.sync_copy(cnt.at[zsl], cnt_b.at[zsl])


def _make_segsum(with_counts):
    mesh = plsc.VectorSubcoreMesh(core_axis_name="c", subcore_axis_name="s")
    out_type = [jax.ShapeDtypeStruct((TBL, HALF), jnp.float32)] * 2
    scratch = [
        pltpu.VMEM((SUPER, 128), jnp.int32),
        pltpu.VMEM((SUPER, 128), jnp.int32),
        pltpu.VMEM((SUPER, 128), jnp.int32),
        pltpu.VMEM((SUPER, 128), jnp.int32),
        pltpu.VMEM((128, HALF), jnp.float32),
        pltpu.VMEM((128, HALF), jnp.float32),
        pltpu.VMEM((128, HALF), jnp.float32),
    ]
    if with_counts:
        out_type = out_type + [jax.ShapeDtypeStruct((TBL, CW), jnp.float32)] * 2
        scratch.append(pltpu.VMEM((128, CW), jnp.float32))
    scratch.append(pltpu.VMEM_SHARED((TBL, HALF), jnp.float32))
    if with_counts:
        scratch.append(pltpu.VMEM_SHARED((TBL, CW), jnp.float32))
    scratch.extend([pltpu.SemaphoreType.DMA] * 6)
    if with_counts:
        scratch.append(pltpu.SemaphoreType.DMA)
    scratch.extend([pltpu.SemaphoreType.DMA] * 2)
    return pl.kernel(
        functools.partial(_seg_body, with_counts),
        out_type=out_type,
        mesh=mesh,
        scratch_types=scratch,
        compiler_params=pltpu.CompilerParams(use_tc_tiling_on_sc=False),
    )


def _norm_body(sl_ref, sh_ref, cnt_ref, lo_ref, hi_ref):
    cnt = jnp.maximum(cnt_ref[...][:, :1], 1.0)
    lo_ref[...] = sl_ref[...] / cnt
    hi_ref[...] = sh_ref[...] / cnt


def _norm(S_lo, S_hi, cnt_col):
    hspec = pl.BlockSpec((1000, HALF), lambda i: (i, 0))
    return pl.pallas_call(
        _norm_body,
        grid=(20,),
        in_specs=[hspec, hspec, pl.BlockSpec((1000, CW), lambda i: (i, 0))],
        out_specs=[hspec, hspec],
        out_shape=[jax.ShapeDtypeStruct((TBL, HALF), jnp.float32)] * 2,
    )(S_lo, S_hi, cnt_col)


def _final_body(xl0, xh0, xl1, xh1, rl0, rh0, rl1, rh1, c0, c1,
                we0, we1, wu0, bu0, wu1, bu1, wm, bm, wih, bih, bhh, wro, bro,
                hn_ref, o_ref):
    f32 = jnp.float32
    xp0 = jnp.concatenate([xl0[...], xh0[...]], axis=1)
    xp1 = jnp.concatenate([xl1[...], xh1[...]], axis=1)
    na0 = jnp.concatenate([rl0[...], rh0[...]], axis=1) / jnp.maximum(c0[...][:, :1], 1.0)
    na1 = jnp.concatenate([rl1[...], rh1[...]], axis=1) / jnp.maximum(c1[...][:, :1], 1.0)
    na0 = jnp.dot(na0, we0[...], preferred_element_type=f32)
    na1 = jnp.dot(na1, we1[...], preferred_element_type=f32)
    wu0v = wu0[...]
    wu1v = wu1[...]
    h0 = jax.nn.relu(jnp.dot(xp0, wu0v[:HID], preferred_element_type=f32)
                     + jnp.dot(na0, wu0v[HID:], preferred_element_type=f32)
                     + bu0[...])
    h1 = jax.nn.relu(jnp.dot(xp1, wu1v[:HID], preferred_element_type=f32)
                     + jnp.dot(na1, wu1v[HID:], preferred_element_type=f32)
                     + bu1[...])
    wmv = wm[...]
    h = jax.nn.relu(jnp.dot(h0, wmv[:HID], preferred_element_type=f32)
                    + jnp.dot(h1, wmv[HID:], preferred_element_type=f32)
                    + bm[...])
    gi = jnp.dot(h, wih[...], preferred_element_type=f32) + bih[...]
    bhhv = bhh[...]
    r = jax.nn.sigmoid(gi[:, :HID] + bhhv[:, :HID])
    z = jax.nn.sigmoid(gi[:, HID:2 * HID] + bhhv[:, HID:2 * HID])
    n = jnp.tanh(gi[:, 2 * HID:] + r * bhhv[:, 2 * HID:])
    hn = (1.0 - z) * n
    hn_ref[...] = hn
    o_ref[...] = jnp.dot(hn, wro[...], preferred_element_type=f32) + bro[...]


def _final(X_lo, X_hi, R_lo, R_hi, cnt_row,
           We0, We1, Wu0, bu0, Wu1, bu1, Wmix, bmix, Wih, bih, bhh, Wro, bro):
    def hblk(off):
        return pl.BlockSpec((1000, HALF), lambda i, o=off: (i + o, 0))

    def cblk(off):
        return pl.BlockSpec((1000, CW), lambda i, o=off: (i + o, 0))

    def whole(a):
        return pl.BlockSpec(a.shape, lambda i: tuple(0 for _ in a.shape))

    ospec = pl.BlockSpec((1000, HID), lambda i: (i, 0))
    return pl.pallas_call(
        _final_body,
        grid=(10,),
        in_specs=[hblk(0), hblk(0), hblk(10), hblk(10),
                  hblk(0), hblk(0), hblk(10), hblk(10),
                  cblk(0), cblk(10),
                  whole(We0), whole(We1),
                  whole(Wu0), whole(bu0), whole(Wu1), whole(bu1),
                  whole(Wmix), whole(bmix), whole(Wih), whole(bih),
                  whole(bhh), whole(Wro), whole(bro)],
        out_specs=[ospec, ospec],
        out_shape=[jax.ShapeDtypeStruct((N_, HID), jnp.float32)] * 2,
    )(X_lo, X_hi, X_lo, X_hi, R_lo, R_hi, R_lo, R_hi, cnt_row, cnt_row,
      We0, We1, Wu0, bu0, Wu1, bu1, Wmix, bmix, Wih, bih, bhh, Wro, bro)


def kernel(x, edge_index, edge_attr, Wv0, We0, Wu0, bu0, Wv1, We1, Wu1, bu1,
           Wmix, bmix, Wih, Whh, bih, bhh, Wro, bro):
    del Whh
    row = edge_index[0].astype(jnp.int32)
    col = edge_index[1].astype(jnp.int32)
    attr = edge_attr.astype(jnp.int32)
    pad = EPAD - E_
    row2 = jnp.pad(row, (0, pad), constant_values=N_).reshape(IDXR, 128)
    col2 = jnp.pad(col, (0, pad), constant_values=N_).reshape(IDXR, 128)
    attr2 = jnp.pad(attr, (0, pad), constant_values=1).reshape(IDXR, 128)

    X_lo, X_hi, g_idx, s_idx = _proj(x, Wv0, Wv1, row2, col2, attr2)

    z64 = jnp.zeros((ZROWS, HALF), jnp.float32)
    zc = jnp.zeros((ZROWS, CW), jnp.float32)
    oc = jnp.ones((128, CW), jnp.float32)

    S_lo, S_hi, cnt_col, cnt_row = _make_segsum(True)(
        X_lo, X_hi, g_idx, s_idx, z64, zc, oc)
    F_lo, F_hi = _norm(S_lo, S_hi, cnt_col)
    R_lo, R_hi = _make_segsum(False)(F_lo, F_hi, s_idx, g_idx, z64)

    hn, o = _final(X_lo, X_hi, R_lo, R_hi, cnt_row, We0, We1,
                   Wu0, bu0.reshape(1, HID), Wu1, bu1.reshape(1, HID),
                   Wmix, bmix.reshape(1, HID), Wih, bih.reshape(1, 3 * HID),
                   bhh.reshape(1, 3 * HID), Wro, bro.reshape(1, HID))
    return hn, o[:, :3]
